# baseline (device time: 29209 ns/iter reference)
import jax
import jax.numpy as jnp
from jax import lax
from jax.experimental import pallas as pl
from jax.experimental.pallas import tpu as pltpu

N_DEV = 4
_GELU_C = 0.7978845608028654


def kernel(x, w_mat):
    m_per, k = x.shape
    _, n_per = w_mat.shape

    def body(x_ref, w_ref, out_ref, comm_ref, send_sems, recv_sems):
        my_pos = lax.axis_index("i")
        left = (my_pos + N_DEV - 1) % N_DEV
        right = (my_pos + 1) % N_DEV

        barrier_sem = pltpu.get_barrier_semaphore()
        for nbr in (left, right):
            pl.semaphore_signal(
                barrier_sem, inc=1,
                device_id=(nbr,), device_id_type=pl.DeviceIdType.MESH,
            )
        pl.semaphore_wait(barrier_sem, 2)

        wb = w_ref[...].astype(jnp.bfloat16)

        def gemm_store(origin, chunk):
            y = jnp.dot(chunk, wb, preferred_element_type=jnp.float32)
            g = 0.5 * y * (1.0 + jnp.tanh(_GELU_C * (y + 0.044715 * y * y * y)))
            out_ref[pl.ds(origin * m_per, m_per), :] = g

        xb = x_ref[...].astype(jnp.bfloat16)
        comm_ref[0] = xb
        gemm_store(my_pos, xb)

        for h in range(N_DEV - 1):
            rdma = pltpu.make_async_remote_copy(
                src_ref=comm_ref.at[h],
                dst_ref=comm_ref.at[h + 1],
                send_sem=send_sems.at[h],
                recv_sem=recv_sems.at[h],
                device_id=(right,),
                device_id_type=pl.DeviceIdType.MESH,
            )
            rdma.start()
            rdma.wait()
            origin = (my_pos + N_DEV - h - 1) % N_DEV
            gemm_store(origin, comm_ref[h + 1])

    return pl.pallas_call(
        body,
        out_shape=jax.ShapeDtypeStruct((N_DEV * m_per, n_per), jnp.float32),
        in_specs=[
            pl.BlockSpec(memory_space=pltpu.VMEM),
            pl.BlockSpec(memory_space=pltpu.VMEM),
        ],
        out_specs=pl.BlockSpec(memory_space=pltpu.VMEM),
        scratch_shapes=[
            pltpu.VMEM((N_DEV, m_per, k), jnp.bfloat16),
            pltpu.SemaphoreType.DMA((N_DEV - 1,)),
            pltpu.SemaphoreType.DMA((N_DEV - 1,)),
        ],
        compiler_params=pltpu.CompilerParams(collective_id=0),
    )(x, w_mat)


# device time: 19235 ns/iter; 1.5185x vs baseline; 1.5185x over previous
import jax
import jax.numpy as jnp
from jax import lax
from jax.experimental import pallas as pl
from jax.experimental.pallas import tpu as pltpu

N_DEV = 4
_GELU_C = 0.7978845608028654


def kernel(x, w_mat):
    m_per, k = x.shape
    _, n_per = w_mat.shape
    half = m_per // 2

    def body(x_ref, w_ref, out_ref, own_ref, l1_ref, r1_ref, d_ref,
             send_sems, recv_sems):
        my_pos = lax.axis_index("i")
        left = (my_pos + N_DEV - 1) % N_DEV
        right = (my_pos + 1) % N_DEV

        barrier_sem = pltpu.get_barrier_semaphore()
        for nbr in (left, right):
            pl.semaphore_signal(
                barrier_sem, inc=1,
                device_id=(nbr,), device_id_type=pl.DeviceIdType.MESH,
            )
        pl.semaphore_wait(barrier_sem, 2)

        wb = w_ref[...].astype(jnp.bfloat16)

        def gemm_store(origin, chunk):
            y = jnp.dot(chunk, wb, preferred_element_type=jnp.float32)
            g = 0.5 * y * (1.0 + jnp.tanh(_GELU_C * (y + 0.044715 * y * y * y)))
            out_ref[pl.ds(origin * m_per, m_per), :] = g

        own_ref[...] = x_ref[...].astype(jnp.bfloat16)

        rdma_r = pltpu.make_async_remote_copy(
            src_ref=own_ref, dst_ref=l1_ref,
            send_sem=send_sems.at[0], recv_sem=recv_sems.at[0],
            device_id=(right,), device_id_type=pl.DeviceIdType.MESH,
        )
        rdma_l = pltpu.make_async_remote_copy(
            src_ref=own_ref, dst_ref=r1_ref,
            send_sem=send_sems.at[1], recv_sem=recv_sems.at[1],
            device_id=(left,), device_id_type=pl.DeviceIdType.MESH,
        )
        rdma_r.start()
        rdma_l.start()

        gemm_store(my_pos, own_ref[...])

        rdma_r.wait_recv()
        rdma_fr = pltpu.make_async_remote_copy(
            src_ref=l1_ref.at[pl.ds(0, half)],
            dst_ref=d_ref.at[pl.ds(0, half)],
            send_sem=send_sems.at[2], recv_sem=recv_sems.at[2],
            device_id=(right,), device_id_type=pl.DeviceIdType.MESH,
        )
        rdma_fr.start()
        gemm_store(left, l1_ref[...])

        rdma_l.wait_recv()
        rdma_fl = pltpu.make_async_remote_copy(
            src_ref=r1_ref.at[pl.ds(half, half)],
            dst_ref=d_ref.at[pl.ds(half, half)],
            send_sem=send_sems.at[3], recv_sem=recv_sems.at[3],
            device_id=(left,), device_id_type=pl.DeviceIdType.MESH,
        )
        rdma_fl.start()
        gemm_store(right, r1_ref[...])

        rdma_fr.wait_recv()
        rdma_fl.wait_recv()
        gemm_store((my_pos + 2) % N_DEV, d_ref[...])

        rdma_r.wait_send()
        rdma_l.wait_send()
        rdma_fr.wait_send()
        rdma_fl.wait_send()

    return pl.pallas_call(
        body,
        out_shape=jax.ShapeDtypeStruct((N_DEV * m_per, n_per), jnp.float32),
        in_specs=[
            pl.BlockSpec(memory_space=pltpu.VMEM),
            pl.BlockSpec(memory_space=pltpu.VMEM),
        ],
        out_specs=pl.BlockSpec(memory_space=pltpu.VMEM),
        scratch_shapes=[
            pltpu.VMEM((m_per, k), jnp.bfloat16),
            pltpu.VMEM((m_per, k), jnp.bfloat16),
            pltpu.VMEM((m_per, k), jnp.bfloat16),
            pltpu.VMEM((m_per, k), jnp.bfloat16),
            pltpu.SemaphoreType.DMA((4,)),
            pltpu.SemaphoreType.DMA((4,)),
        ],
        compiler_params=pltpu.CompilerParams(collective_id=0),
    )(x, w_mat)


# device time: 16356 ns/iter; 1.7858x vs baseline; 1.1760x over previous
import jax
import jax.numpy as jnp
from jax import lax
from jax.experimental import pallas as pl
from jax.experimental.pallas import tpu as pltpu

N_DEV = 4
_GELU_C = 0.7978845608028654


def kernel(x, w_mat):
    m_per, k = x.shape
    _, n_per = w_mat.shape
    half = m_per // 2

    def body(x_ref, w_ref, out_ref, own_ref, l1_ref, r1_ref, d_ref,
             send_sems, recv_sems):
        my_pos = lax.axis_index("i")
        left = (my_pos + N_DEV - 1) % N_DEV
        right = (my_pos + 1) % N_DEV

        barrier_sem = pltpu.get_barrier_semaphore()
        for nbr in (left, right):
            pl.semaphore_signal(
                barrier_sem, inc=1,
                device_id=(nbr,), device_id_type=pl.DeviceIdType.MESH,
            )
        pl.semaphore_wait(barrier_sem, 2)

        wb = w_ref[...].astype(jnp.bfloat16)
        own_ref[...] = x_ref[...].astype(jnp.bfloat16)

        def rc(src, dst, sem, dev):
            return pltpu.make_async_remote_copy(
                src_ref=src, dst_ref=dst,
                send_sem=send_sems.at[sem], recv_sem=recv_sems.at[sem],
                device_id=(dev,), device_id_type=pl.DeviceIdType.MESH,
            )

        lo = pl.ds(0, half)
        hi = pl.ds(half, half)
        s_r_lo = rc(own_ref.at[lo], l1_ref.at[lo], 0, right)
        s_r_hi = rc(own_ref.at[hi], l1_ref.at[hi], 1, right)
        s_l_hi = rc(own_ref.at[hi], r1_ref.at[hi], 2, left)
        s_l_lo = rc(own_ref.at[lo], r1_ref.at[lo], 3, left)
        s_r_lo.start()
        s_l_hi.start()
        s_r_hi.start()
        s_l_lo.start()

        def gemm_store(origin, chunk):
            y = jnp.dot(chunk, wb, preferred_element_type=jnp.float32)
            g = 0.5 * y * (1.0 + jnp.tanh(_GELU_C * (y + 0.044715 * y * y * y)))
            out_ref[pl.ds(origin * m_per, m_per), :] = g

        gemm_store(my_pos, own_ref[...])

        s_r_lo.wait_recv()
        f_r = rc(l1_ref.at[lo], d_ref.at[lo], 4, right)
        f_r.start()
        s_l_hi.wait_recv()
        f_l = rc(r1_ref.at[hi], d_ref.at[hi], 5, left)
        f_l.start()

        s_r_hi.wait_recv()
        gemm_store(left, l1_ref[...])
        s_l_lo.wait_recv()
        gemm_store(right, r1_ref[...])

        f_r.wait_recv()
        f_l.wait_recv()
        gemm_store((my_pos + 2) % N_DEV, d_ref[...])

        for r in (s_r_lo, s_r_hi, s_l_hi, s_l_lo, f_r, f_l):
            r.wait_send()

    return pl.pallas_call(
        body,
        out_shape=jax.ShapeDtypeStruct((N_DEV * m_per, n_per), jnp.float32),
        in_specs=[
            pl.BlockSpec(memory_space=pltpu.VMEM),
            pl.BlockSpec(memory_space=pltpu.VMEM),
        ],
        out_specs=pl.BlockSpec(memory_space=pltpu.VMEM),
        scratch_shapes=[
            pltpu.VMEM((m_per, k), jnp.bfloat16),
            pltpu.VMEM((m_per, k), jnp.bfloat16),
            pltpu.VMEM((m_per, k), jnp.bfloat16),
            pltpu.VMEM((m_per, k), jnp.bfloat16),
            pltpu.SemaphoreType.DMA((6,)),
            pltpu.SemaphoreType.DMA((6,)),
        ],
        compiler_params=pltpu.CompilerParams(collective_id=0),
    )(x, w_mat)
